# Initial kernel scaffold; baseline (speedup 1.0000x reference)
import jax
import jax.numpy as jnp
from jax.experimental import pallas as pl


def kernel(p2, p3, p4, p5, p6, conv0_w, conv0_b, conv1_w, conv1_b, obj_w, obj_b, delta_w, delta_b, image_size):
    raise NotImplementedError("write your pallas kernel here")


# speedup vs baseline: 5.0121x; 5.0121x over previous
import math

import jax
import jax.numpy as jnp
from jax.experimental import pallas as pl
from jax.experimental.pallas import tpu as pltpu

SIZES = ((32,), (64,), (128,), (256,), (512,))
ASPECT_RATIOS = (0.5, 1.0, 2.0)
STRIDES = (4, 8, 16, 32, 64)
PRE_NMS_TOPK = 2000
POST_NMS_TOPK = 1000
NMS_THRESH = 0.7
N = PRE_NMS_TOPK
BLK = 125
NBLK = N // BLK


def _cell_anchors_k(sizes, aspect_ratios):
    anchors = []
    for s in sizes:
        area = s * s
        for ar in aspect_ratios:
            w = math.sqrt(area * ar)
            h = area / w
            anchors.append([-w / 2.0, -h / 2.0, w / 2.0, h / 2.0])
    return jnp.asarray(anchors, dtype=jnp.float32)


def _grid_anchors_k(H, W, stride, cell):
    sx = jnp.arange(W, dtype=jnp.float32) * stride + stride / 2.0
    sy = jnp.arange(H, dtype=jnp.float32) * stride + stride / 2.0
    yy, xx = jnp.meshgrid(sy, sx, indexing='ij')
    shifts = jnp.stack([xx, yy, xx, yy], axis=-1).reshape(-1, 1, 4)
    return (shifts + cell[None]).reshape(-1, 4)


def _decode_boxes_k(deltas, anchors):
    ax = (anchors[:, 0] + anchors[:, 2]) * 0.5
    ay = (anchors[:, 1] + anchors[:, 3]) * 0.5
    aw = anchors[:, 2] - anchors[:, 0]
    ah = anchors[:, 3] - anchors[:, 1]
    dx = deltas[:, 0]
    dy = deltas[:, 1]
    clamp = math.log(1000.0 / 16)
    dw = jnp.minimum(deltas[:, 2], clamp)
    dh = jnp.minimum(deltas[:, 3], clamp)
    cx = dx * aw + ax
    cy = dy * ah + ay
    w = jnp.exp(dw) * aw
    h = jnp.exp(dh) * ah
    return jnp.stack([cx - w / 2.0, cy - h / 2.0,
                      cx + w / 2.0, cy + h / 2.0], axis=1)


def _conv2d_k(x, w, b):
    y = jax.lax.conv_general_dilated(
        x, w, (1, 1), 'SAME', dimension_numbers=('NCHW', 'OIHW', 'NCHW'))
    return y + b[None, :, None, None]


def _sel_body(k_ref, p_ref, o_ref):
    keep = k_ref[0]
    u = jax.lax.broadcasted_iota(jnp.int32, (BLK, BLK), 0)
    v = jax.lax.broadcasted_iota(jnp.int32, (BLK, BLK), 1)
    tri = (u < v).astype(jnp.float32)
    pieces = []
    carry = jnp.zeros((1, 1), jnp.float32)
    for i in range(NBLK):
        kb_i = keep[0:1, i * BLK:(i + 1) * BLK]
        pieces.append(jnp.dot(kb_i, tri,
                              preferred_element_type=jnp.float32) + carry)
        carry = carry + jnp.sum(kb_i).reshape(1, 1)
    pos = jnp.concatenate(pieces, axis=1)
    count = carry[0, 0]

    jrow = jax.lax.broadcasted_iota(
        jnp.int32, (POST_NMS_TOPK, 1), 0).astype(jnp.float32)
    lane = jax.lax.broadcasted_iota(jnp.int32, (1, N), 1)
    sel = (pos == jrow).astype(jnp.float32) * keep
    fill = (jrow >= count).astype(jnp.float32) * (lane == 0).astype(jnp.float32)
    sel = sel + fill
    o_ref[...] = jnp.dot(sel, p_ref[0],
                         preferred_element_type=jnp.float32,
                         precision=jax.lax.Precision.HIGHEST)[None]


def _select_topk(keep_f, p_top):
    B = p_top.shape[0]
    return pl.pallas_call(
        _sel_body,
        grid=(B,),
        in_specs=[
            pl.BlockSpec((1, 1, N), lambda bb: (bb, 0, 0)),
            pl.BlockSpec((1, N, 4), lambda bb: (bb, 0, 0)),
        ],
        out_specs=pl.BlockSpec((1, POST_NMS_TOPK, 4), lambda bb: (bb, 0, 0)),
        out_shape=jax.ShapeDtypeStruct((B, POST_NMS_TOPK, 4), jnp.float32),
    )(keep_f, p_top)


def _nms_keep_k(boxes, thresh):
    b = jax.lax.stop_gradient(boxes)
    x1, y1, x2, y2 = b[:, 0], b[:, 1], b[:, 2], b[:, 3]
    areas = (x2 - x1) * (y2 - y1)
    xx1 = jnp.maximum(x1[:, None], x1[None, :])
    yy1 = jnp.maximum(y1[:, None], y1[None, :])
    xx2 = jnp.minimum(x2[:, None], x2[None, :])
    yy2 = jnp.minimum(y2[:, None], y2[None, :])
    inter = jnp.maximum(xx2 - xx1, 0.0) * jnp.maximum(yy2 - yy1, 0.0)
    iou = inter / (areas[:, None] + areas[None, :] - inter + 1e-9)
    over = iou > thresh
    n = b.shape[0]
    ar = jnp.arange(n)

    def body(i, keep):
        sup = over[i] & (ar > i)
        return jnp.where(keep[i], keep & jnp.logical_not(sup), keep)

    return jax.lax.fori_loop(0, n, body, jnp.ones((n,), dtype=bool))


def _sup_body(p_ref, o_ref):
    p_all = p_ref[0]
    eye4 = (jax.lax.broadcasted_iota(jnp.int32, (4, 4), 0) ==
            jax.lax.broadcasted_iota(jnp.int32, (4, 4), 1)
            ).astype(jnp.float32)
    pT = jax.lax.dot_general(
        eye4, p_all, (((1,), (1,)), ((), ())),
        preferred_element_type=jnp.float32,
        precision=jax.lax.Precision.HIGHEST)
    x1r = pT[0:1]
    y1r = pT[1:2]
    x2r = pT[2:3]
    y2r = pT[3:4]
    arear = (x2r - x1r) * (y2r - y1r)

    keep = jnp.ones((1, N), jnp.float32)
    laneb = jax.lax.broadcasted_iota(jnp.int32, (1, BLK), 1)
    for blk in range(NBLK):
        start = blk * BLK
        p_blk = p_all[start:start + BLK, :]
        x1c = p_blk[:, 0:1]
        y1c = p_blk[:, 1:2]
        x2c = p_blk[:, 2:3]
        y2c = p_blk[:, 3:4]
        areac = (x2c - x1c) * (y2c - y1c)

        xx1 = jnp.maximum(x1c, x1r)
        yy1 = jnp.maximum(y1c, y1r)
        xx2 = jnp.minimum(x2c, x2r)
        yy2 = jnp.minimum(y2c, y2r)
        inter = jnp.maximum(xx2 - xx1, 0.0) * jnp.maximum(yy2 - yy1, 0.0)
        iou = inter / (areac + arear - inter + 1e-9)
        over = (iou > NMS_THRESH).astype(jnp.float32)

        over_i = over[:, start:start + BLK]
        kb = keep[0:1, start:start + BLK]
        for r in range(BLK):
            onehot = (laneb == r).astype(jnp.float32)
            kr = jnp.sum(kb * onehot)
            tail = (laneb > r).astype(jnp.float32)
            sup = over_i[r:r + 1, :] * tail
            kb = jnp.where(kr > 0.0, kb * (1.0 - sup), kb)
        pieces = []
        if start > 0:
            pieces.append(keep[:, :start])
        pieces.append(kb)
        if start + BLK < N:
            pieces.append(keep[:, start + BLK:])
        keep = jnp.concatenate(pieces, axis=1) if len(pieces) > 1 else kb

        rowg = jax.lax.broadcasted_iota(jnp.int32, (BLK, N), 0) + start
        colg = jax.lax.broadcasted_iota(jnp.int32, (BLK, N), 1)
        m = over * (colg > rowg).astype(jnp.float32)
        sup_all = jnp.dot(kb, m, preferred_element_type=jnp.float32)
        keep = keep * (1.0 - jnp.minimum(sup_all, 1.0))

    o_ref[...] = (keep > 0.5)[None]


def _nms_suppress(p_top):
    B = p_top.shape[0]
    out = pl.pallas_call(
        _sup_body,
        grid=(B,),
        in_specs=[pl.BlockSpec((1, N, 4), lambda bb: (bb, 0, 0))],
        out_specs=pl.BlockSpec((1, 1, N), lambda bb: (bb, 0, 0)),
        out_shape=jax.ShapeDtypeStruct((B, 1, N), jnp.bool_),
    )(p_top)
    return out[:, 0, :]


def kernel(p2, p3, p4, p5, p6, conv0_w, conv0_b, conv1_w, conv1_b,
           obj_w, obj_b, delta_w, delta_b, image_size):
    feats = [p2, p3, p4, p5, p6]
    props = []
    scrs = []
    for feat, stride, sizes in zip(feats, STRIDES, SIZES):
        t = jax.nn.relu(_conv2d_k(feat, conv0_w, conv0_b))
        t = jax.nn.relu(_conv2d_k(t, conv1_w, conv1_b))
        obj = _conv2d_k(t, obj_w, obj_b)
        delta = _conv2d_k(t, delta_w, delta_b)
        B, A, H, W = obj.shape
        cell = _cell_anchors_k(sizes, ASPECT_RATIOS)
        anch = _grid_anchors_k(H, W, stride, cell)
        obj_flat = jnp.transpose(obj, (0, 2, 3, 1)).reshape(B, -1)
        delta_flat = jnp.transpose(delta, (0, 2, 3, 1)).reshape(B, -1, 4)
        anch_b = jnp.broadcast_to(anch[None], (B,) + anch.shape).reshape(-1, 4)
        boxes = _decode_boxes_k(delta_flat.reshape(-1, 4), anch_b).reshape(B, -1, 4)
        props.append(boxes)
        scrs.append(obj_flat)
    proposals = jnp.concatenate(props, axis=1)
    scores = jnp.concatenate(scrs, axis=1)
    Himg = jnp.asarray(image_size, dtype=jnp.float32)
    Wimg = jnp.asarray(image_size, dtype=jnp.float32)

    ps = []
    for bidx in range(proposals.shape[0]):
        p = proposals[bidx]
        p = jnp.stack([jnp.clip(p[:, 0], 0.0, Wimg),
                       jnp.clip(p[:, 1], 0.0, Himg),
                       jnp.clip(p[:, 2], 0.0, Wimg),
                       jnp.clip(p[:, 3], 0.0, Himg)], axis=1)
        s = jax.nn.sigmoid(scores[bidx])
        k = min(PRE_NMS_TOPK, s.shape[0])
        sv, si = jax.lax.top_k(s, k)
        ps.append(p[si])

    p_top = jnp.stack(ps)
    keep = _nms_suppress(p_top)

    results = []
    for bidx in range(proposals.shape[0]):
        idx = jnp.nonzero(keep[bidx], size=POST_NMS_TOPK, fill_value=0)[0]
        results.append(ps[bidx][idx])
    return jnp.stack(results)
